# baseline (speedup 1.0000x reference)
import jax
import jax.numpy as jnp
from jax import lax
from jax.experimental import pallas as pl
from jax.experimental.pallas import tpu as pltpu
from jax.experimental.pallas import tpu_sc as plsc

NC, NS = 2, 16
NW = NC * NS
COORD_DIM = 85
SPATIAL_DIM = 256
ROW_DIM = 3 * COORD_DIM + 3 * SPATIAL_DIM
CHUNK = 16
PAD_DIM = 96
SEGS = ((0, 0, PAD_DIM), (1, 85, PAD_DIM), (2, 170, PAD_DIM),
        (3, 255, SPATIAL_DIM), (4, 511, SPATIAL_DIM), (5, 767, SPATIAL_DIM))


def _body(coords, angs, tss, x_tab, y_tab, z_tab, a_tab, e_tab, t_tab,
          out, cslab, aslab, vidx, bufs, abufs, gsems, wsems):
    wid = lax.axis_index("s") * NC + lax.axis_index("c")
    tabs = (x_tab, y_tab, z_tab, a_tab, e_tab, t_tab)
    vix, viy, viz, via, vie, vit = vidx
    rpw = vix.shape[0]
    base = wid * rpw
    nch = rpw // CHUNK
    pltpu.sync_copy(coords.at[pl.ds(base * 3, rpw * 3)], cslab)
    pltpu.sync_copy(angs.at[pl.ds(base * 2, rpw * 2)], aslab)
    pltpu.sync_copy(tss.at[pl.ds(base, rpw)], vit)
    iota = lax.iota(jnp.int32, 16)
    iota3 = iota * 3
    iota2 = iota * 2
    consts = [iota + 16 * i for i in range(SPATIAL_DIM // 16)]
    tail_mask = iota < (COORD_DIM - 80)

    @pl.loop(0, rpw // 16)
    def _ext(r):
        b3 = r * 48
        b2 = r * 32
        s = pl.ds(16 * r, 16)
        vix[s] = plsc.load_gather(cslab, [b3 + iota3])
        viy[s] = plsc.load_gather(cslab, [b3 + iota3 + 1])
        viz[s] = plsc.load_gather(cslab, [b3 + iota3 + 2])
        via[s] = plsc.load_gather(aslab, [b2 + iota2])
        vie[s] = plsc.load_gather(aslab, [b2 + iota2 + 1])

    def issue_gathers(p, c):
        off = c * CHUNK
        for k in range(6):
            pltpu.async_copy(tabs[k].at[vidx[k].at[pl.ds(off, CHUNK)]],
                             bufs[p][k], gsems[p])

    def drain_gathers(p):
        for k in range(6):
            pltpu.make_async_copy(tabs[k].at[vidx[k].at[pl.ds(0, CHUNK)]],
                                  bufs[p][k], gsems[p]).wait()

    def assemble(p):
        abuf = abufs[p]

        @plsc.parallel_loop(0, CHUNK, 1, unroll=1)
        def _row(r):
            rbase = r * ROW_DIM
            ops = []
            for k, o, w in SEGS:
                sbv = lax.broadcast(rbase + o, (16,))
                buf = bufs[p][k]
                for i in range(w // 16):
                    ops.append((buf, 16 * i, sbv + consts[i]))
            for g in range(0, len(ops), 8):
                grp = ops[g:g + 8]
                vs = [buf[r, pl.ds(c0, 16)] for buf, c0, _ in grp]
                for (_, _, dst), v in zip(grp, vs):
                    plsc.store_scatter(abuf, [dst], v)

    def write_out(p, c):
        pltpu.async_copy(
            abufs[p],
            out.at[pl.ds((base + c * CHUNK) * ROW_DIM, CHUNK * ROW_DIM)],
            wsems[p])

    def wait_write(p):
        pltpu.make_async_copy(
            abufs[p], out.at[pl.ds(0, CHUNK * ROW_DIM)], wsems[p]).wait()

    issue_gathers(0, 0)

    @pl.loop(0, nch, step=2)
    def _pair(c):
        issue_gathers(1, c + 1)
        drain_gathers(0)

        @pl.when(c > 0)
        def _():
            wait_write(0)
        assemble(0)
        write_out(0, c)
        issue_gathers(0, lax.min(c + 2, nch - 1))
        drain_gathers(1)

        @pl.when(c > 0)
        def _():
            wait_write(1)
        assemble(1)
        write_out(1, c + 1)

    drain_gathers(0)
    wait_write(0)
    wait_write(1)


def kernel(batch_size, num_views, coordinates, angles, timestamps,
           x_emb, y_emb, z_emb, az_emb, el_emb, t_emb):
    b, v = coordinates.shape[0], coordinates.shape[1]
    n = b * v
    rpw = n // NW
    coords = coordinates.reshape(-1)
    angs = angles.reshape(-1)
    tss = timestamps.reshape(-1)
    pad = ((0, 0), (0, PAD_DIM - COORD_DIM))
    x_p, y_p, z_p = (jnp.pad(t, pad) for t in (x_emb, y_emb, z_emb))

    bufset = (
        pltpu.VMEM((CHUNK, PAD_DIM), jnp.float32),
        pltpu.VMEM((CHUNK, PAD_DIM), jnp.float32),
        pltpu.VMEM((CHUNK, PAD_DIM), jnp.float32),
        pltpu.VMEM((CHUNK, SPATIAL_DIM), jnp.float32),
        pltpu.VMEM((CHUNK, SPATIAL_DIM), jnp.float32),
        pltpu.VMEM((CHUNK, SPATIAL_DIM), jnp.float32),
    )
    run = pl.kernel(
        _body,
        out_type=jax.ShapeDtypeStruct((n * ROW_DIM,), jnp.float32),
        mesh=plsc.VectorSubcoreMesh(
            core_axis_name="c", subcore_axis_name="s",
            num_cores=NC, num_subcores=NS),
        compiler_params=pltpu.CompilerParams(
            use_tc_tiling_on_sc=False, needs_layout_passes=False),
        scratch_types=[
            pltpu.VMEM((rpw * 3,), jnp.int32),
            pltpu.VMEM((rpw * 2,), jnp.int32),
            tuple(pltpu.VMEM((rpw,), jnp.int32) for _ in range(6)),
            (bufset, bufset),
            tuple(pltpu.VMEM((CHUNK * ROW_DIM,), jnp.float32)
                  for _ in range(2)),
            (pltpu.SemaphoreType.DMA, pltpu.SemaphoreType.DMA),
            (pltpu.SemaphoreType.DMA, pltpu.SemaphoreType.DMA),
        ],
    )
    flat = run(coords, angs, tss, x_p, y_p, z_p, az_emb, el_emb, t_emb)
    return flat.reshape(b, v, ROW_DIM)
